# Initial kernel scaffold; baseline (speedup 1.0000x reference)
import jax
import jax.numpy as jnp
from jax.experimental import pallas as pl


def kernel(x, edge_index, W0, att_src0, att_dst0, b0, A0, B0, al_src0, al_dst0, bl0, W1, att_src1, att_dst1, b1, A1, B1, al_src1, al_dst1, bl1):
    raise NotImplementedError("write your pallas kernel here")


# speedup vs baseline: 11.0610x; 11.0610x over previous
import functools

import jax
import jax.numpy as jnp
from jax import lax
from jax.experimental import pallas as pl
from jax.experimental.pallas import tpu as pltpu
from jax.experimental.pallas import tpu_sc as plsc

N = 10000
E = 320000
NPAD = 10240
DUMP = N
EP = 331776
NS = 16
W = 128
CH = EP // NS
NWIN = CH // W
RPS = NPAD // NS
LANES = 16
DC0 = 128
DC1 = 128
RB = 1024
GRID = NPAD // RB

_MESH = plsc.VectorSubcoreMesh(core_axis_name="c", subcore_axis_name="s")
_GDN = lax.GatherDimensionNumbers(offset_dims=(), collapsed_slice_dims=(0,),
                                  start_index_map=(0,))


def _zero_spmem(sid, rows, zden, accs, dens):
    zeros16 = jnp.zeros((LANES,), jnp.float32)
    dc = rows.shape[1]

    def _zrow(i, c):
        for j in range(dc // LANES):
            rows[i, pl.ds(j * LANES, LANES)] = zeros16
        return c
    lax.fori_loop(0, W, _zrow, 0)

    def _zden(i, c):
        zden[pl.ds(i * LANES, LANES)] = zeros16
        return c
    lax.fori_loop(0, RPS // LANES, _zden, 0)

    for acc in accs:
        for t in range(RPS // W):
            r0 = sid * RPS + t * W
            pltpu.sync_copy(rows, acc.at[pl.ds(r0, W)])
    for den in dens:
        pltpu.sync_copy(zden, den.at[pl.ds(sid * RPS, RPS)])


def _edge_window(sid, w, src_h, dst_h, srcv, dstv):
    base = pl.multiple_of(sid * CH + w * W, W)
    pltpu.sync_copy(src_h.at[pl.ds(base, W)], srcv)
    pltpu.sync_copy(dst_h.at[pl.ds(base, W)], dstv)


def _edge_exp(ss_h, sd_h, srcv, dstv, g1, g2, ev, sem):
    pltpu.async_copy(ss_h.at[srcv], g1, sem).wait()
    pltpu.async_copy(sd_h.at[dstv], g2, sem).wait()
    for k in range(W // LANES):
        sl = pl.ds(k * LANES, LANES)
        a = g1[sl] + g2[sl]
        a = jnp.where(a > 0, a, 0.2 * a)
        ev[sl] = jnp.exp(a)


def _coef_div(ev, dv):
    for k in range(W // LANES):
        sl = pl.ds(k * LANES, LANES)
        ev[sl] = ev[sl] / (dv[sl] + 1e-16)


def _splat(c16, l):
    idxl = jnp.full((LANES, 1), l, jnp.int32)
    return lax.gather(c16, idxl, _GDN, slice_sizes=(1,),
                      mode=lax.GatherScatterMode.PROMISE_IN_BOUNDS)


def _sc_layer0_body(src_h, dst_h, s0_h, s1_h, s2_h, s3_h,
                    hb0_h, hb1_h, hl0_h, hl1_h, out0_h, out1_h,
                    srcv, dstv, g1, g2, ebv, elv, dbv, dlv,
                    rowsb, rowsl, zden, sem, acc0, denb, denl):
    cid = lax.axis_index("c")
    sid = lax.axis_index("s")

    _zero_spmem(sid, rowsb, zden, (acc0,), (denb, denl))
    plsc.subcore_barrier()

    def _p1(w, c):
        _edge_window(sid, w, src_h, dst_h, srcv, dstv)
        for (ss_h, sd_h, ev, den) in ((s0_h, s1_h, ebv, denb),
                                      (s2_h, s3_h, elv, denl)):
            _edge_exp(ss_h, sd_h, srcv, dstv, g1, g2, ev, sem)
            pltpu.sync_copy(ev, den.at[dstv], add=True)
        return c
    lax.fori_loop(0, NWIN, _p1, 0)
    plsc.subcore_barrier()

    def _phase23(hb_h, hl_h, out_h):
        def _p2(w, c):
            _edge_window(sid, w, src_h, dst_h, srcv, dstv)
            for (ss_h, sd_h, ev, den, dv) in ((s0_h, s1_h, ebv, denb, dbv),
                                              (s2_h, s3_h, elv, denl, dlv)):
                _edge_exp(ss_h, sd_h, srcv, dstv, g1, g2, ev, sem)
                pltpu.async_copy(den.at[dstv], dv, sem).wait()
                _coef_div(ev, dv)
            pltpu.async_copy(hb_h.at[srcv], rowsb, sem).wait()
            pltpu.async_copy(hl_h.at[srcv], rowsl, sem).wait()

            def _scale(k, c2):
                cb16 = ebv[pl.ds(k * LANES, LANES)]
                cl16 = elv[pl.ds(k * LANES, LANES)]
                for l in range(LANES):
                    cb = _splat(cb16, l)
                    cl = _splat(cl16, l)
                    r = k * LANES + l
                    for j in range(DC0 // LANES):
                        js = pl.ds(j * LANES, LANES)
                        rowsb[r, js] = rowsb[r, js] * cb + rowsl[r, js] * cl
                return c2
            lax.fori_loop(0, W // LANES, _scale, 0)
            pltpu.sync_copy(rowsb, acc0.at[dstv], add=True)
            return c
        lax.fori_loop(0, NWIN, _p2, 0)
        plsc.subcore_barrier()
        for t in range(RPS // W):
            r0 = sid * RPS + t * W
            pltpu.sync_copy(acc0.at[pl.ds(r0, W)], out_h.at[pl.ds(r0, W)])

    @pl.when(cid == 0)
    def _():
        _phase23(hb0_h, hl0_h, out0_h)

    @pl.when(cid == 1)
    def _():
        _phase23(hb1_h, hl1_h, out1_h)


def _sc_layer1_body(src_h, dst_h, s0_h, s1_h, s2_h, s3_h, hb_h, hl_h,
                    o1_h, o2_h,
                    srcv, dstv, g1, g2, ev, dv,
                    rows, zden, sem, acc, den):
    cid = lax.axis_index("c")
    sid = lax.axis_index("s")

    def _branch(ss_h, sd_h, h_h, out_h):
        _zero_spmem(sid, rows, zden, (acc,), (den,))
        plsc.subcore_barrier()

        def _p1(w, c):
            _edge_window(sid, w, src_h, dst_h, srcv, dstv)
            _edge_exp(ss_h, sd_h, srcv, dstv, g1, g2, ev, sem)
            pltpu.sync_copy(ev, den.at[dstv], add=True)
            return c
        lax.fori_loop(0, NWIN, _p1, 0)
        plsc.subcore_barrier()

        def _p2(w, c):
            _edge_window(sid, w, src_h, dst_h, srcv, dstv)
            _edge_exp(ss_h, sd_h, srcv, dstv, g1, g2, ev, sem)
            pltpu.async_copy(den.at[dstv], dv, sem).wait()
            _coef_div(ev, dv)
            pltpu.async_copy(h_h.at[srcv], rows, sem).wait()

            def _scale(k, c2):
                c16 = ev[pl.ds(k * LANES, LANES)]
                for l in range(LANES):
                    cc = _splat(c16, l)
                    r = k * LANES + l
                    for j in range(DC1 // LANES):
                        js = pl.ds(j * LANES, LANES)
                        rows[r, js] = rows[r, js] * cc
                return c2
            lax.fori_loop(0, W // LANES, _scale, 0)
            pltpu.sync_copy(rows, acc.at[dstv], add=True)
            return c
        lax.fori_loop(0, NWIN, _p2, 0)
        plsc.subcore_barrier()
        for t in range(RPS // W):
            r0 = sid * RPS + t * W
            pltpu.sync_copy(acc.at[pl.ds(r0, W)], out_h.at[pl.ds(r0, W)])

    @pl.when(cid == 0)
    def _():
        _branch(s0_h, s1_h, hb_h, o1_h)

    @pl.when(cid == 1)
    def _():
        _branch(s2_h, s3_h, hl_h, o2_h)


def _sc_layer0(src, dst, s0, s1, s2, s3, hb0, hb1, hl0, hl1):
    out_type = [jax.ShapeDtypeStruct((NPAD, DC0), jnp.float32)] * 2
    scratch = [
        pltpu.VMEM((W,), jnp.int32),
        pltpu.VMEM((W,), jnp.int32),
        pltpu.VMEM((W,), jnp.float32),
        pltpu.VMEM((W,), jnp.float32),
        pltpu.VMEM((W,), jnp.float32),
        pltpu.VMEM((W,), jnp.float32),
        pltpu.VMEM((W,), jnp.float32),
        pltpu.VMEM((W,), jnp.float32),
        pltpu.VMEM((W, DC0), jnp.float32),
        pltpu.VMEM((W, DC0), jnp.float32),
        pltpu.VMEM((RPS,), jnp.float32),
        pltpu.SemaphoreType.DMA,
        pltpu.VMEM_SHARED((NPAD, DC0), jnp.float32),
        pltpu.VMEM_SHARED((NPAD,), jnp.float32),
        pltpu.VMEM_SHARED((NPAD,), jnp.float32),
    ]
    return pl.kernel(
        _sc_layer0_body,
        out_type,
        mesh=_MESH,
        scratch_types=scratch,
        name="sc_gat_layer0",
    )(src, dst, s0, s1, s2, s3, hb0, hb1, hl0, hl1)


def _sc_layer1(src, dst, s0, s1, s2, s3, hb, hl):
    out_type = [jax.ShapeDtypeStruct((NPAD, DC1), jnp.float32)] * 2
    scratch = [
        pltpu.VMEM((W,), jnp.int32),
        pltpu.VMEM((W,), jnp.int32),
        pltpu.VMEM((W,), jnp.float32),
        pltpu.VMEM((W,), jnp.float32),
        pltpu.VMEM((W,), jnp.float32),
        pltpu.VMEM((W,), jnp.float32),
        pltpu.VMEM((W, DC1), jnp.float32),
        pltpu.VMEM((RPS,), jnp.float32),
        pltpu.SemaphoreType.DMA,
        pltpu.VMEM_SHARED((NPAD, DC1), jnp.float32),
        pltpu.VMEM_SHARED((NPAD,), jnp.float32),
    ]
    return pl.kernel(
        _sc_layer1_body,
        out_type,
        mesh=_MESH,
        scratch_types=scratch,
        name="sc_gat_layer1",
    )(src, dst, s0, s1, s2, s3, hb, hl)


def _tc_layer0_body(x_ref, w0t_ref, a0t_ref, b0t_ref, attb_ref, attl_ref,
                    hb0_ref, hb1_ref, hl0_ref, hl1_ref, scal_ref):
    x = x_ref[...]
    hb = jnp.dot(x, w0t_ref[...], preferred_element_type=jnp.float32)
    hl = jnp.dot(jnp.dot(x, a0t_ref[...], preferred_element_type=jnp.float32),
                 b0t_ref[...], preferred_element_type=jnp.float32)
    hb0_ref[...] = hb[:, :128]
    hb1_ref[...] = hb[:, 128:]
    hl0_ref[...] = hl[:, :128]
    hl1_ref[...] = hl[:, 128:]
    sb = lax.dot_general(attb_ref[...], hb, (((0,), (1,)), ((), ())),
                         preferred_element_type=jnp.float32)
    sl = lax.dot_general(attl_ref[...], hl, (((0,), (1,)), ((), ())),
                         preferred_element_type=jnp.float32)
    scal_ref[...] = jnp.concatenate([sb, sl], axis=0)


def _tc_layer0(xp, w0t, a0t, b0t, attb, attl):
    return pl.pallas_call(
        _tc_layer0_body,
        grid=(GRID,),
        in_specs=[
            pl.BlockSpec((RB, 128), lambda i: (i, 0)),
            pl.BlockSpec((128, 256), lambda i: (0, 0)),
            pl.BlockSpec((128, 32), lambda i: (0, 0)),
            pl.BlockSpec((32, 256), lambda i: (0, 0)),
            pl.BlockSpec((256, 2), lambda i: (0, 0)),
            pl.BlockSpec((256, 2), lambda i: (0, 0)),
        ],
        out_specs=[
            pl.BlockSpec((RB, 128), lambda i: (i, 0)),
            pl.BlockSpec((RB, 128), lambda i: (i, 0)),
            pl.BlockSpec((RB, 128), lambda i: (i, 0)),
            pl.BlockSpec((RB, 128), lambda i: (i, 0)),
            pl.BlockSpec((4, RB), lambda i: (0, i)),
        ],
        out_shape=[
            jax.ShapeDtypeStruct((NPAD, 128), jnp.float32),
            jax.ShapeDtypeStruct((NPAD, 128), jnp.float32),
            jax.ShapeDtypeStruct((NPAD, 128), jnp.float32),
            jax.ShapeDtypeStruct((NPAD, 128), jnp.float32),
            jax.ShapeDtypeStruct((4, NPAD), jnp.float32),
        ],
    )(xp, w0t, a0t, b0t, attb, attl)


def _tc_layer1_body(m0_ref, m1_ref, bias_ref, w1t_ref, a1t_ref, b1t_ref,
                    attb_ref, attl_ref,
                    hb_ref, hl_ref, scal_ref):
    x1 = jnp.concatenate([m0_ref[...], m1_ref[...]], axis=1) + bias_ref[...]
    hb = jnp.dot(x1, w1t_ref[...], preferred_element_type=jnp.float32)
    hl = jnp.dot(jnp.dot(x1, a1t_ref[...], preferred_element_type=jnp.float32),
                 b1t_ref[...], preferred_element_type=jnp.float32)
    hb_ref[...] = hb
    hl_ref[...] = hl
    sb = lax.dot_general(attb_ref[...], hb, (((0,), (1,)), ((), ())),
                         preferred_element_type=jnp.float32)
    sl = lax.dot_general(attl_ref[...], hl, (((0,), (1,)), ((), ())),
                         preferred_element_type=jnp.float32)
    scal_ref[...] = jnp.concatenate([sb, sl], axis=0)


def _tc_layer1(m0, m1, bias, w1t, a1t, b1t, attb, attl):
    return pl.pallas_call(
        _tc_layer1_body,
        grid=(GRID,),
        in_specs=[
            pl.BlockSpec((RB, 128), lambda i: (i, 0)),
            pl.BlockSpec((RB, 128), lambda i: (i, 0)),
            pl.BlockSpec((1, 256), lambda i: (0, 0)),
            pl.BlockSpec((256, 128), lambda i: (0, 0)),
            pl.BlockSpec((256, 32), lambda i: (0, 0)),
            pl.BlockSpec((32, 128), lambda i: (0, 0)),
            pl.BlockSpec((128, 2), lambda i: (0, 0)),
            pl.BlockSpec((128, 2), lambda i: (0, 0)),
        ],
        out_specs=[
            pl.BlockSpec((RB, 128), lambda i: (i, 0)),
            pl.BlockSpec((RB, 128), lambda i: (i, 0)),
            pl.BlockSpec((4, RB), lambda i: (0, i)),
        ],
        out_shape=[
            jax.ShapeDtypeStruct((NPAD, 128), jnp.float32),
            jax.ShapeDtypeStruct((NPAD, 128), jnp.float32),
            jax.ShapeDtypeStruct((4, NPAD), jnp.float32),
        ],
    )(m0, m1, bias, w1t, a1t, b1t, attb, attl)


def _tc_final_body(o1_ref, o2_ref, b1_ref, bl1_ref,
                   osum_ref, e1_ref, e2_ref):
    e1 = o1_ref[...] + b1_ref[...]
    e2 = o2_ref[...] + bl1_ref[...]
    e1_ref[...] = e1
    e2_ref[...] = e2
    osum_ref[...] = e1 + e2


def _tc_final(o1, o2, b1, bl1):
    fb = 1000
    return pl.pallas_call(
        _tc_final_body,
        grid=(N // fb,),
        in_specs=[
            pl.BlockSpec((fb, 128), lambda i: (i, 0)),
            pl.BlockSpec((fb, 128), lambda i: (i, 0)),
            pl.BlockSpec((1, 128), lambda i: (0, 0)),
            pl.BlockSpec((1, 128), lambda i: (0, 0)),
        ],
        out_specs=[
            pl.BlockSpec((fb, 128), lambda i: (i, 0)),
            pl.BlockSpec((fb, 128), lambda i: (i, 0)),
            pl.BlockSpec((fb, 128), lambda i: (i, 0)),
        ],
        out_shape=[
            jax.ShapeDtypeStruct((N, 128), jnp.float32),
            jax.ShapeDtypeStruct((N, 128), jnp.float32),
            jax.ShapeDtypeStruct((N, 128), jnp.float32),
        ],
    )(o1, o2, b1, bl1)


def kernel(x, edge_index, W0, att_src0, att_dst0, b0, A0, B0, al_src0,
           al_dst0, bl0, W1, att_src1, att_dst1, b1, A1, B1, al_src1,
           al_dst1, bl1):
    loop = jnp.arange(N, dtype=jnp.int32)
    npad_e = EP - E - N
    src = jnp.concatenate([edge_index[0], loop,
                           jnp.zeros((npad_e,), jnp.int32)])
    dst = jnp.concatenate([edge_index[1], loop,
                           jnp.full((npad_e,), DUMP, jnp.int32)])
    xp = jnp.pad(x, ((0, NPAD - N), (0, 0)))

    attb0 = jnp.stack([att_src0, att_dst0], axis=1)
    attl0 = jnp.stack([al_src0, al_dst0], axis=1)
    attb1 = jnp.stack([att_src1, att_dst1], axis=1)
    attl1 = jnp.stack([al_src1, al_dst1], axis=1)

    hb0, hb1, hl0, hl1, scal0 = _tc_layer0(xp, W0.T, A0.T, B0.T, attb0, attl0)
    m0, m1 = _sc_layer0(src, dst, scal0[0], scal0[1], scal0[2], scal0[3],
                        hb0, hb1, hl0, hl1)

    bias0 = (b0 + bl0)[None, :]
    h1b, h1l, scal1 = _tc_layer1(m0, m1, bias0, W1.T, A1.T, B1.T, attb1, attl1)
    o1, o2 = _sc_layer1(src, dst, scal1[0], scal1[1], scal1[2], scal1[3],
                        h1b, h1l)

    osum, emb1, emb2 = _tc_final(o1, o2, b1[None, :], bl1[None, :])
    return (osum, emb1, emb2)


# speedup vs baseline: 14.5165x; 1.3124x over previous
import functools

import jax
import jax.numpy as jnp
from jax import lax
from jax.experimental import pallas as pl
from jax.experimental.pallas import tpu as pltpu
from jax.experimental.pallas import tpu_sc as plsc

N = 10000
E = 320000
NPAD = 10240
DUMP = N
EP = 331776
NS = 16
W = 128
CH = EP // NS
NWIN = CH // W
RPS = NPAD // NS
LANES = 16
DC0 = 128
DC1 = 128
RB = 1024
GRID = NPAD // RB

_MESH = plsc.VectorSubcoreMesh(core_axis_name="c", subcore_axis_name="s")
_GDN = lax.GatherDimensionNumbers(offset_dims=(), collapsed_slice_dims=(0,),
                                  start_index_map=(0,))


def _zero_spmem(sid, rows, zden, accs, dens):
    zeros16 = jnp.zeros((LANES,), jnp.float32)
    dc = rows.shape[1]

    def _zrow(i, c):
        for j in range(dc // LANES):
            rows[i, pl.ds(j * LANES, LANES)] = zeros16
        return c
    lax.fori_loop(0, W, _zrow, 0)

    def _zden(i, c):
        zden[pl.ds(i * LANES, LANES)] = zeros16
        return c
    lax.fori_loop(0, RPS // LANES, _zden, 0)

    for acc in accs:
        for t in range(RPS // W):
            r0 = sid * RPS + t * W
            pltpu.sync_copy(rows, acc.at[pl.ds(r0, W)])
    for den in dens:
        pltpu.sync_copy(zden, den.at[pl.ds(sid * RPS, RPS)])


def _edge_window(sid, w, src_h, dst_h, srcv, dstv):
    base = pl.multiple_of(sid * CH + w * W, W)
    pltpu.sync_copy(src_h.at[pl.ds(base, W)], srcv)
    pltpu.sync_copy(dst_h.at[pl.ds(base, W)], dstv)


def _edge_exp(ss_h, sd_h, srcv, dstv, g1, g2, ev, sem):
    pltpu.async_copy(ss_h.at[srcv], g1, sem).wait()
    pltpu.async_copy(sd_h.at[dstv], g2, sem).wait()
    for k in range(W // LANES):
        sl = pl.ds(k * LANES, LANES)
        a = g1[sl] + g2[sl]
        a = jnp.where(a > 0, a, 0.2 * a)
        ev[sl] = jnp.exp(a)


def _splat(c16, l):
    idxl = jnp.full((LANES, 1), l, jnp.int32)
    return lax.gather(c16, idxl, _GDN, slice_sizes=(1,),
                      mode=lax.GatherScatterMode.PROMISE_IN_BOUNDS)


def _msg_pass(sid, src_h, dst_h, ss_h, sd_h, h_h,
              srcv, dstv, g1, g2, ev, rows, sem, acc, den):
    def _p(w, c):
        _edge_window(sid, w, src_h, dst_h, srcv, dstv)
        _edge_exp(ss_h, sd_h, srcv, dstv, g1, g2, ev, sem)
        pltpu.sync_copy(ev, den.at[dstv], add=True)
        pltpu.async_copy(h_h.at[srcv], rows, sem).wait()

        def _scale(k, c2):
            c16 = ev[pl.ds(k * LANES, LANES)]
            for l in range(LANES):
                cc = _splat(c16, l)
                r = k * LANES + l
                for j in range(DC0 // LANES):
                    js = pl.ds(j * LANES, LANES)
                    rows[r, js] = rows[r, js] * cc
            return c2
        lax.fori_loop(0, W // LANES, _scale, 0)
        pltpu.sync_copy(rows, acc.at[dstv], add=True)
        return c
    lax.fori_loop(0, NWIN, _p, 0)


def _writeback_norm(sid, acc, den, rows, dv, out_h, sem):
    for t in range(RPS // W):
        r0 = sid * RPS + t * W
        pltpu.sync_copy(acc.at[pl.ds(r0, W)], rows)
        pltpu.sync_copy(den.at[pl.ds(r0, W)], dv)

        def _norm(k, c):
            d16 = dv[pl.ds(k * LANES, LANES)]
            inv16 = 1.0 / (d16 + 1e-16)
            for l in range(LANES):
                iv = _splat(inv16, l)
                r = k * LANES + l
                for j in range(DC0 // LANES):
                    js = pl.ds(j * LANES, LANES)
                    rows[r, js] = rows[r, js] * iv
            return c
        lax.fori_loop(0, W // LANES, _norm, 0)
        pltpu.sync_copy(rows, out_h.at[pl.ds(r0, W)])


def _sc_pass_body(src_h, dst_h, sA_h, dA_h, hA_h, sB_h, dB_h, hB_h,
                  oA_h, oB_h,
                  srcv, dstv, g1, g2, ev, dv, rows, zden, sem, acc, den):
    cid = lax.axis_index("c")
    sid = lax.axis_index("s")

    _zero_spmem(sid, rows, zden, (acc,), (den,))
    plsc.subcore_barrier()

    def _run(ss_h, sd_h, h_h, out_h):
        _msg_pass(sid, src_h, dst_h, ss_h, sd_h, h_h,
                  srcv, dstv, g1, g2, ev, rows, sem, acc, den)
        plsc.subcore_barrier()
        _writeback_norm(sid, acc, den, rows, dv, out_h, sem)

    @pl.when(cid == 0)
    def _():
        _run(sA_h, dA_h, hA_h, oA_h)

    @pl.when(cid == 1)
    def _():
        _run(sB_h, dB_h, hB_h, oB_h)


def _sc_pass(src, dst, sA, dA, hA, sB, dB, hB):
    out_type = [jax.ShapeDtypeStruct((NPAD, DC0), jnp.float32)] * 2
    scratch = [
        pltpu.VMEM((W,), jnp.int32),
        pltpu.VMEM((W,), jnp.int32),
        pltpu.VMEM((W,), jnp.float32),
        pltpu.VMEM((W,), jnp.float32),
        pltpu.VMEM((W,), jnp.float32),
        pltpu.VMEM((W,), jnp.float32),
        pltpu.VMEM((W, DC0), jnp.float32),
        pltpu.VMEM((RPS,), jnp.float32),
        pltpu.SemaphoreType.DMA,
        pltpu.VMEM_SHARED((NPAD, DC0), jnp.float32),
        pltpu.VMEM_SHARED((NPAD,), jnp.float32),
    ]
    return pl.kernel(
        _sc_pass_body,
        out_type,
        mesh=_MESH,
        scratch_types=scratch,
        name="sc_gat_pass",
    )(src, dst, sA, dA, hA, sB, dB, hB)


def _tc_layer0_body(x_ref, w0t_ref, a0t_ref, b0t_ref, attb_ref, attl_ref,
                    hb0_ref, hb1_ref, hl0_ref, hl1_ref, scal_ref):
    x = x_ref[...]
    hb = jnp.dot(x, w0t_ref[...], preferred_element_type=jnp.float32)
    hl = jnp.dot(jnp.dot(x, a0t_ref[...], preferred_element_type=jnp.float32),
                 b0t_ref[...], preferred_element_type=jnp.float32)
    hb0_ref[...] = hb[:, :128]
    hb1_ref[...] = hb[:, 128:]
    hl0_ref[...] = hl[:, :128]
    hl1_ref[...] = hl[:, 128:]
    sb = lax.dot_general(attb_ref[...], hb, (((0,), (1,)), ((), ())),
                         preferred_element_type=jnp.float32)
    sl = lax.dot_general(attl_ref[...], hl, (((0,), (1,)), ((), ())),
                         preferred_element_type=jnp.float32)
    scal_ref[...] = jnp.concatenate([sb, sl], axis=0)


def _tc_layer0(xp, w0t, a0t, b0t, attb, attl):
    return pl.pallas_call(
        _tc_layer0_body,
        grid=(GRID,),
        in_specs=[
            pl.BlockSpec((RB, 128), lambda i: (i, 0)),
            pl.BlockSpec((128, 256), lambda i: (0, 0)),
            pl.BlockSpec((128, 32), lambda i: (0, 0)),
            pl.BlockSpec((32, 256), lambda i: (0, 0)),
            pl.BlockSpec((256, 2), lambda i: (0, 0)),
            pl.BlockSpec((256, 2), lambda i: (0, 0)),
        ],
        out_specs=[
            pl.BlockSpec((RB, 128), lambda i: (i, 0)),
            pl.BlockSpec((RB, 128), lambda i: (i, 0)),
            pl.BlockSpec((RB, 128), lambda i: (i, 0)),
            pl.BlockSpec((RB, 128), lambda i: (i, 0)),
            pl.BlockSpec((4, RB), lambda i: (0, i)),
        ],
        out_shape=[
            jax.ShapeDtypeStruct((NPAD, 128), jnp.float32),
            jax.ShapeDtypeStruct((NPAD, 128), jnp.float32),
            jax.ShapeDtypeStruct((NPAD, 128), jnp.float32),
            jax.ShapeDtypeStruct((NPAD, 128), jnp.float32),
            jax.ShapeDtypeStruct((4, NPAD), jnp.float32),
        ],
    )(xp, w0t, a0t, b0t, attb, attl)


def _tc_layer1_body(mb0_ref, mb1_ref, ml0_ref, ml1_ref, bias_ref,
                    w1t_ref, a1t_ref, b1t_ref, attb_ref, attl_ref,
                    hb_ref, hl_ref, scal_ref):
    x1 = (jnp.concatenate([mb0_ref[...], mb1_ref[...]], axis=1)
          + jnp.concatenate([ml0_ref[...], ml1_ref[...]], axis=1)
          + bias_ref[...])
    hb = jnp.dot(x1, w1t_ref[...], preferred_element_type=jnp.float32)
    hl = jnp.dot(jnp.dot(x1, a1t_ref[...], preferred_element_type=jnp.float32),
                 b1t_ref[...], preferred_element_type=jnp.float32)
    hb_ref[...] = hb
    hl_ref[...] = hl
    sb = lax.dot_general(attb_ref[...], hb, (((0,), (1,)), ((), ())),
                         preferred_element_type=jnp.float32)
    sl = lax.dot_general(attl_ref[...], hl, (((0,), (1,)), ((), ())),
                         preferred_element_type=jnp.float32)
    scal_ref[...] = jnp.concatenate([sb, sl], axis=0)


def _tc_layer1(mb0, mb1, ml0, ml1, bias, w1t, a1t, b1t, attb, attl):
    return pl.pallas_call(
        _tc_layer1_body,
        grid=(GRID,),
        in_specs=[
            pl.BlockSpec((RB, 128), lambda i: (i, 0)),
            pl.BlockSpec((RB, 128), lambda i: (i, 0)),
            pl.BlockSpec((RB, 128), lambda i: (i, 0)),
            pl.BlockSpec((RB, 128), lambda i: (i, 0)),
            pl.BlockSpec((1, 256), lambda i: (0, 0)),
            pl.BlockSpec((256, 128), lambda i: (0, 0)),
            pl.BlockSpec((256, 32), lambda i: (0, 0)),
            pl.BlockSpec((32, 128), lambda i: (0, 0)),
            pl.BlockSpec((128, 2), lambda i: (0, 0)),
            pl.BlockSpec((128, 2), lambda i: (0, 0)),
        ],
        out_specs=[
            pl.BlockSpec((RB, 128), lambda i: (i, 0)),
            pl.BlockSpec((RB, 128), lambda i: (i, 0)),
            pl.BlockSpec((4, RB), lambda i: (0, i)),
        ],
        out_shape=[
            jax.ShapeDtypeStruct((NPAD, 128), jnp.float32),
            jax.ShapeDtypeStruct((NPAD, 128), jnp.float32),
            jax.ShapeDtypeStruct((4, NPAD), jnp.float32),
        ],
    )(mb0, mb1, ml0, ml1, bias, w1t, a1t, b1t, attb, attl)


def _tc_final_body(o1_ref, o2_ref, b1_ref, bl1_ref,
                   osum_ref, e1_ref, e2_ref):
    e1 = o1_ref[...] + b1_ref[...]
    e2 = o2_ref[...] + bl1_ref[...]
    e1_ref[...] = e1
    e2_ref[...] = e2
    osum_ref[...] = e1 + e2


def _tc_final(o1, o2, b1, bl1):
    fb = 1000
    return pl.pallas_call(
        _tc_final_body,
        grid=(N // fb,),
        in_specs=[
            pl.BlockSpec((fb, 128), lambda i: (i, 0)),
            pl.BlockSpec((fb, 128), lambda i: (i, 0)),
            pl.BlockSpec((1, 128), lambda i: (0, 0)),
            pl.BlockSpec((1, 128), lambda i: (0, 0)),
        ],
        out_specs=[
            pl.BlockSpec((fb, 128), lambda i: (i, 0)),
            pl.BlockSpec((fb, 128), lambda i: (i, 0)),
            pl.BlockSpec((fb, 128), lambda i: (i, 0)),
        ],
        out_shape=[
            jax.ShapeDtypeStruct((N, 128), jnp.float32),
            jax.ShapeDtypeStruct((N, 128), jnp.float32),
            jax.ShapeDtypeStruct((N, 128), jnp.float32),
        ],
    )(o1, o2, b1, bl1)


def kernel(x, edge_index, W0, att_src0, att_dst0, b0, A0, B0, al_src0,
           al_dst0, bl0, W1, att_src1, att_dst1, b1, A1, B1, al_src1,
           al_dst1, bl1):
    loop = jnp.arange(N, dtype=jnp.int32)
    npad_e = EP - E - N
    src = jnp.concatenate([edge_index[0], loop,
                           jnp.zeros((npad_e,), jnp.int32)])
    dst = jnp.concatenate([edge_index[1], loop,
                           jnp.full((npad_e,), DUMP, jnp.int32)])
    xp = jnp.pad(x, ((0, NPAD - N), (0, 0)))

    attb0 = jnp.stack([att_src0, att_dst0], axis=1)
    attl0 = jnp.stack([al_src0, al_dst0], axis=1)
    attb1 = jnp.stack([att_src1, att_dst1], axis=1)
    attl1 = jnp.stack([al_src1, al_dst1], axis=1)

    hb0, hb1, hl0, hl1, scal0 = _tc_layer0(xp, W0.T, A0.T, B0.T, attb0, attl0)
    mb0, mb1 = _sc_pass(src, dst, scal0[0], scal0[1], hb0,
                        scal0[0], scal0[1], hb1)
    ml0, ml1 = _sc_pass(src, dst, scal0[2], scal0[3], hl0,
                        scal0[2], scal0[3], hl1)

    bias0 = (b0 + bl0)[None, :]
    h1b, h1l, scal1 = _tc_layer1(mb0, mb1, ml0, ml1, bias0,
                                 W1.T, A1.T, B1.T, attb1, attl1)
    o1, o2 = _sc_pass(src, dst, scal1[0], scal1[1], h1b,
                      scal1[2], scal1[3], h1l)

    osum, emb1, emb2 = _tc_final(o1, o2, b1[None, :], bl1[None, :])
    return (osum, emb1, emb2)


# speedup vs baseline: 15.6192x; 1.0760x over previous
import functools

import jax
import jax.numpy as jnp
from jax import lax
from jax.experimental import pallas as pl
from jax.experimental.pallas import tpu as pltpu
from jax.experimental.pallas import tpu_sc as plsc

N = 10000
E = 320000
NPAD = 10240
DUMP = N
EP = 331776
NS = 16
W = 128
CH = EP // NS
NWIN = CH // W
RPS = NPAD // NS
LANES = 16
DC0 = 128
DC1 = 128
RB = 1024
GRID = NPAD // RB

_MESH = plsc.VectorSubcoreMesh(core_axis_name="c", subcore_axis_name="s")
_GDN = lax.GatherDimensionNumbers(offset_dims=(), collapsed_slice_dims=(0,),
                                  start_index_map=(0,))


def _zero_spmem(sid, rows, zden, accs, dens):
    zeros16 = jnp.zeros((LANES,), jnp.float32)
    dc = rows.shape[1]

    def _zrow(i, c):
        for j in range(dc // LANES):
            rows[i, pl.ds(j * LANES, LANES)] = zeros16
        return c
    lax.fori_loop(0, W, _zrow, 0)

    def _zden(i, c):
        zden[pl.ds(i * LANES, LANES)] = zeros16
        return c
    lax.fori_loop(0, RPS // LANES, _zden, 0)

    for acc in accs:
        for t in range(RPS // W):
            r0 = sid * RPS + t * W
            pltpu.sync_copy(rows, acc.at[pl.ds(r0, W)])
    for den in dens:
        pltpu.sync_copy(zden, den.at[pl.ds(sid * RPS, RPS)])


def _edge_window(sid, w, sd_h, sdv, srcv, dstv):
    base = pl.multiple_of((sid * NWIN + w) * 2 * W, 2 * W)
    pltpu.sync_copy(sd_h.at[pl.ds(base, 2 * W)], sdv)
    for k in range(W // LANES):
        sl = pl.ds(k * LANES, LANES)
        srcv[sl] = sdv[sl]
        dstv[sl] = sdv[pl.ds(W + k * LANES, LANES)]


def _edge_exp(ss_h, sd_h, srcv, dstv, g1, g2, ev, sem):
    pltpu.async_copy(ss_h.at[srcv], g1, sem).wait()
    pltpu.async_copy(sd_h.at[dstv], g2, sem).wait()
    for k in range(W // LANES):
        sl = pl.ds(k * LANES, LANES)
        a = g1[sl] + g2[sl]
        a = jnp.where(a > 0, a, 0.2 * a)
        ev[sl] = jnp.exp(a)


def _splat(c16, l):
    idxl = jnp.full((LANES, 1), l, jnp.int32)
    return lax.gather(c16, idxl, _GDN, slice_sizes=(1,),
                      mode=lax.GatherScatterMode.PROMISE_IN_BOUNDS)


def _msg_pass(sid, sd_h2, ss_h, sd_h, h_h,
              sdv, srcv, dstv, g1, g2, ev, rows, sem, acc, den):
    def _p(w, c):
        _edge_window(sid, w, sd_h2, sdv, srcv, dstv)
        _edge_exp(ss_h, sd_h, srcv, dstv, g1, g2, ev, sem)
        pltpu.sync_copy(ev, den.at[dstv], add=True)
        pltpu.async_copy(h_h.at[srcv], rows, sem).wait()

        def _scale(k, c2):
            c16 = ev[pl.ds(k * LANES, LANES)]
            for l in range(LANES):
                cc = _splat(c16, l)
                r = k * LANES + l
                for j in range(DC0 // LANES):
                    js = pl.ds(j * LANES, LANES)
                    rows[r, js] = rows[r, js] * cc
            return c2
        lax.fori_loop(0, W // LANES, _scale, 0)
        pltpu.sync_copy(rows, acc.at[dstv], add=True)
        return c
    lax.fori_loop(0, NWIN, _p, 0)


def _writeback_norm(sid, acc, den, rows, dv, out_h, sem):
    for t in range(RPS // W):
        r0 = sid * RPS + t * W
        pltpu.sync_copy(acc.at[pl.ds(r0, W)], rows)
        pltpu.sync_copy(den.at[pl.ds(r0, W)], dv)

        def _norm(k, c):
            d16 = dv[pl.ds(k * LANES, LANES)]
            inv16 = 1.0 / (d16 + 1e-16)
            for l in range(LANES):
                iv = _splat(inv16, l)
                r = k * LANES + l
                for j in range(DC0 // LANES):
                    js = pl.ds(j * LANES, LANES)
                    rows[r, js] = rows[r, js] * iv
            return c
        lax.fori_loop(0, W // LANES, _norm, 0)
        pltpu.sync_copy(rows, out_h.at[pl.ds(r0, W)])


def _sc_pass_body(sd_h2, sA_h, dA_h, hA_h, sB_h, dB_h, hB_h,
                  oA_h, oB_h,
                  sdv, srcv, dstv, g1, g2, ev, dv, rows, zden, sem,
                  acc, den):
    cid = lax.axis_index("c")
    sid = lax.axis_index("s")

    _zero_spmem(sid, rows, zden, (acc,), (den,))
    plsc.subcore_barrier()

    def _run(ss_h, sd_h, h_h, out_h):
        _msg_pass(sid, sd_h2, ss_h, sd_h, h_h,
                  sdv, srcv, dstv, g1, g2, ev, rows, sem, acc, den)
        plsc.subcore_barrier()
        _writeback_norm(sid, acc, den, rows, dv, out_h, sem)

    @pl.when(cid == 0)
    def _():
        _run(sA_h, dA_h, hA_h, oA_h)

    @pl.when(cid == 1)
    def _():
        _run(sB_h, dB_h, hB_h, oB_h)


def _sc_pass(srcdst, sA, dA, hA, sB, dB, hB):
    out_type = [jax.ShapeDtypeStruct((NPAD, DC0), jnp.float32)] * 2
    scratch = [
        pltpu.VMEM((2 * W,), jnp.int32),
        pltpu.VMEM((W,), jnp.int32),
        pltpu.VMEM((W,), jnp.int32),
        pltpu.VMEM((W,), jnp.float32),
        pltpu.VMEM((W,), jnp.float32),
        pltpu.VMEM((W,), jnp.float32),
        pltpu.VMEM((W,), jnp.float32),
        pltpu.VMEM((W, DC0), jnp.float32),
        pltpu.VMEM((RPS,), jnp.float32),
        pltpu.SemaphoreType.DMA,
        pltpu.VMEM_SHARED((NPAD, DC0), jnp.float32),
        pltpu.VMEM_SHARED((NPAD,), jnp.float32),
    ]
    return pl.kernel(
        _sc_pass_body,
        out_type,
        mesh=_MESH,
        scratch_types=scratch,
        name="sc_gat_pass",
    )(srcdst, sA, dA, hA, sB, dB, hB)


def _tc_layer0_body(x_ref, w0t_ref, a0t_ref, b0t_ref, attb_ref, attl_ref,
                    hb0_ref, hb1_ref, hl0_ref, hl1_ref, scal_ref):
    x = x_ref[...]
    hb = jnp.dot(x, w0t_ref[...], preferred_element_type=jnp.float32)
    hl = jnp.dot(jnp.dot(x, a0t_ref[...], preferred_element_type=jnp.float32),
                 b0t_ref[...], preferred_element_type=jnp.float32)
    hb0_ref[...] = hb[:, :128]
    hb1_ref[...] = hb[:, 128:]
    hl0_ref[...] = hl[:, :128]
    hl1_ref[...] = hl[:, 128:]
    sb = lax.dot_general(attb_ref[...], hb, (((0,), (1,)), ((), ())),
                         preferred_element_type=jnp.float32)
    sl = lax.dot_general(attl_ref[...], hl, (((0,), (1,)), ((), ())),
                         preferred_element_type=jnp.float32)
    scal_ref[...] = jnp.concatenate([sb, sl], axis=0)


def _tc_layer0(xp, w0t, a0t, b0t, attb, attl):
    return pl.pallas_call(
        _tc_layer0_body,
        grid=(GRID,),
        in_specs=[
            pl.BlockSpec((RB, 128), lambda i: (i, 0)),
            pl.BlockSpec((128, 256), lambda i: (0, 0)),
            pl.BlockSpec((128, 32), lambda i: (0, 0)),
            pl.BlockSpec((32, 256), lambda i: (0, 0)),
            pl.BlockSpec((256, 2), lambda i: (0, 0)),
            pl.BlockSpec((256, 2), lambda i: (0, 0)),
        ],
        out_specs=[
            pl.BlockSpec((RB, 128), lambda i: (i, 0)),
            pl.BlockSpec((RB, 128), lambda i: (i, 0)),
            pl.BlockSpec((RB, 128), lambda i: (i, 0)),
            pl.BlockSpec((RB, 128), lambda i: (i, 0)),
            pl.BlockSpec((4, RB), lambda i: (0, i)),
        ],
        out_shape=[
            jax.ShapeDtypeStruct((NPAD, 128), jnp.float32),
            jax.ShapeDtypeStruct((NPAD, 128), jnp.float32),
            jax.ShapeDtypeStruct((NPAD, 128), jnp.float32),
            jax.ShapeDtypeStruct((NPAD, 128), jnp.float32),
            jax.ShapeDtypeStruct((4, NPAD), jnp.float32),
        ],
    )(xp, w0t, a0t, b0t, attb, attl)


def _tc_layer1_body(mb0_ref, mb1_ref, ml0_ref, ml1_ref, bias_ref,
                    w1t_ref, a1t_ref, b1t_ref, attb_ref, attl_ref,
                    hb_ref, hl_ref, scal_ref):
    x1 = (jnp.concatenate([mb0_ref[...], mb1_ref[...]], axis=1)
          + jnp.concatenate([ml0_ref[...], ml1_ref[...]], axis=1)
          + bias_ref[...])
    hb = jnp.dot(x1, w1t_ref[...], preferred_element_type=jnp.float32)
    hl = jnp.dot(jnp.dot(x1, a1t_ref[...], preferred_element_type=jnp.float32),
                 b1t_ref[...], preferred_element_type=jnp.float32)
    hb_ref[...] = hb
    hl_ref[...] = hl
    sb = lax.dot_general(attb_ref[...], hb, (((0,), (1,)), ((), ())),
                         preferred_element_type=jnp.float32)
    sl = lax.dot_general(attl_ref[...], hl, (((0,), (1,)), ((), ())),
                         preferred_element_type=jnp.float32)
    scal_ref[...] = jnp.concatenate([sb, sl], axis=0)


def _tc_layer1(mb0, mb1, ml0, ml1, bias, w1t, a1t, b1t, attb, attl):
    return pl.pallas_call(
        _tc_layer1_body,
        grid=(GRID,),
        in_specs=[
            pl.BlockSpec((RB, 128), lambda i: (i, 0)),
            pl.BlockSpec((RB, 128), lambda i: (i, 0)),
            pl.BlockSpec((RB, 128), lambda i: (i, 0)),
            pl.BlockSpec((RB, 128), lambda i: (i, 0)),
            pl.BlockSpec((1, 256), lambda i: (0, 0)),
            pl.BlockSpec((256, 128), lambda i: (0, 0)),
            pl.BlockSpec((256, 32), lambda i: (0, 0)),
            pl.BlockSpec((32, 128), lambda i: (0, 0)),
            pl.BlockSpec((128, 2), lambda i: (0, 0)),
            pl.BlockSpec((128, 2), lambda i: (0, 0)),
        ],
        out_specs=[
            pl.BlockSpec((RB, 128), lambda i: (i, 0)),
            pl.BlockSpec((RB, 128), lambda i: (i, 0)),
            pl.BlockSpec((4, RB), lambda i: (0, i)),
        ],
        out_shape=[
            jax.ShapeDtypeStruct((NPAD, 128), jnp.float32),
            jax.ShapeDtypeStruct((NPAD, 128), jnp.float32),
            jax.ShapeDtypeStruct((4, NPAD), jnp.float32),
        ],
    )(mb0, mb1, ml0, ml1, bias, w1t, a1t, b1t, attb, attl)


def _tc_final_body(o1_ref, o2_ref, b1_ref, bl1_ref,
                   osum_ref, e1_ref, e2_ref):
    e1 = o1_ref[...] + b1_ref[...]
    e2 = o2_ref[...] + bl1_ref[...]
    e1_ref[...] = e1
    e2_ref[...] = e2
    osum_ref[...] = e1 + e2


def _tc_final(o1, o2, b1, bl1):
    fb = 1000
    return pl.pallas_call(
        _tc_final_body,
        grid=(N // fb,),
        in_specs=[
            pl.BlockSpec((fb, 128), lambda i: (i, 0)),
            pl.BlockSpec((fb, 128), lambda i: (i, 0)),
            pl.BlockSpec((1, 128), lambda i: (0, 0)),
            pl.BlockSpec((1, 128), lambda i: (0, 0)),
        ],
        out_specs=[
            pl.BlockSpec((fb, 128), lambda i: (i, 0)),
            pl.BlockSpec((fb, 128), lambda i: (i, 0)),
            pl.BlockSpec((fb, 128), lambda i: (i, 0)),
        ],
        out_shape=[
            jax.ShapeDtypeStruct((N, 128), jnp.float32),
            jax.ShapeDtypeStruct((N, 128), jnp.float32),
            jax.ShapeDtypeStruct((N, 128), jnp.float32),
        ],
    )(o1, o2, b1, bl1)


def kernel(x, edge_index, W0, att_src0, att_dst0, b0, A0, B0, al_src0,
           al_dst0, bl0, W1, att_src1, att_dst1, b1, A1, B1, al_src1,
           al_dst1, bl1):
    loop = jnp.arange(N, dtype=jnp.int32)
    npad_e = EP - E - N
    src = jnp.concatenate([edge_index[0], loop,
                           jnp.zeros((npad_e,), jnp.int32)])
    dst = jnp.concatenate([edge_index[1], loop,
                           jnp.full((npad_e,), DUMP, jnp.int32)])
    srcdst = jnp.stack([src.reshape(EP // W, W),
                        dst.reshape(EP // W, W)], axis=1).reshape(2 * EP)
    xp = jnp.pad(x, ((0, NPAD - N), (0, 0)))

    attb0 = jnp.stack([att_src0, att_dst0], axis=1)
    attl0 = jnp.stack([al_src0, al_dst0], axis=1)
    attb1 = jnp.stack([att_src1, att_dst1], axis=1)
    attl1 = jnp.stack([al_src1, al_dst1], axis=1)

    hb0, hb1, hl0, hl1, scal0 = _tc_layer0(xp, W0.T, A0.T, B0.T, attb0, attl0)
    mb0, mb1 = _sc_pass(srcdst, scal0[0], scal0[1], hb0,
                        scal0[0], scal0[1], hb1)
    ml0, ml1 = _sc_pass(srcdst, scal0[2], scal0[3], hl0,
                        scal0[2], scal0[3], hl1)

    bias0 = (b0 + bl0)[None, :]
    h1b, h1l, scal1 = _tc_layer1(mb0, mb1, ml0, ml1, bias0,
                                 W1.T, A1.T, B1.T, attb1, attl1)
    o1, o2 = _sc_pass(srcdst, scal1[0], scal1[1], h1b,
                      scal1[2], scal1[3], h1l)

    osum, emb1, emb2 = _tc_final(o1, o2, b1[None, :], bl1[None, :])
    return (osum, emb1, emb2)


# speedup vs baseline: 15.6896x; 1.0045x over previous
import functools

import jax
import jax.numpy as jnp
from jax import lax
from jax.experimental import pallas as pl
from jax.experimental.pallas import tpu as pltpu
from jax.experimental.pallas import tpu_sc as plsc

N = 10000
E = 320000
NPAD = 10240
DUMP = N
EP = 331776
NS = 16
W = 128
CH = EP // NS
NWIN = CH // W
RPS = NPAD // NS
LANES = 16
DC0 = 128
DC1 = 128
RB = 1024
GRID = NPAD // RB

_MESH = plsc.VectorSubcoreMesh(core_axis_name="c", subcore_axis_name="s")
_GDN = lax.GatherDimensionNumbers(offset_dims=(), collapsed_slice_dims=(0,),
                                  start_index_map=(0,))


def _zero_spmem(sid, rows, zden, accs, dens):
    zeros16 = jnp.zeros((LANES,), jnp.float32)
    dc = rows.shape[1]

    def _zrow(i, c):
        for j in range(dc // LANES):
            rows[i, pl.ds(j * LANES, LANES)] = zeros16
        return c
    lax.fori_loop(0, W, _zrow, 0)

    def _zden(i, c):
        zden[pl.ds(i * LANES, LANES)] = zeros16
        return c
    lax.fori_loop(0, RPS // LANES, _zden, 0)

    for acc in accs:
        for t in range(RPS // W):
            r0 = sid * RPS + t * W
            pltpu.sync_copy(rows, acc.at[pl.ds(r0, W)])
    for den in dens:
        pltpu.sync_copy(zden, den.at[pl.ds(sid * RPS, RPS)])


def _edge_window(sid, w, sd_h, sdv, srcv, dstv):
    base = pl.multiple_of((sid * NWIN + w) * 2 * W, 2 * W)
    pltpu.sync_copy(sd_h.at[pl.ds(base, 2 * W)], sdv)
    for k in range(W // LANES):
        sl = pl.ds(k * LANES, LANES)
        srcv[sl] = sdv[sl]
        dstv[sl] = sdv[pl.ds(W + k * LANES, LANES)]


def _edge_exp(ss_h, sd_h, srcv, dstv, g1, g2, ev, sem):
    pltpu.async_copy(ss_h.at[srcv], g1, sem).wait()
    pltpu.async_copy(sd_h.at[dstv], g2, sem).wait()
    for k in range(W // LANES):
        sl = pl.ds(k * LANES, LANES)
        a = g1[sl] + g2[sl]
        a = jnp.where(a > 0, a, 0.2 * a)
        ev[sl] = jnp.exp(a)


def _splat(c16, l):
    idxl = jnp.full((LANES, 1), l, jnp.int32)
    return lax.gather(c16, idxl, _GDN, slice_sizes=(1,),
                      mode=lax.GatherScatterMode.PROMISE_IN_BOUNDS)


def _msg_pass(sid, sd_h2, ss_h, sd_h, h_h,
              sdv, srcv, dstv, g1, g2, ev, rows, sem, acc, den):
    def _p(w, c):
        _edge_window(sid, w, sd_h2, sdv, srcv, dstv)
        _edge_exp(ss_h, sd_h, srcv, dstv, g1, g2, ev, sem)
        pltpu.sync_copy(ev, den.at[dstv], add=True)
        pltpu.async_copy(h_h.at[srcv], rows, sem).wait()

        def _scale(k, c2):
            c16 = ev[pl.ds(k * LANES, LANES)]
            for l in range(LANES):
                cc = _splat(c16, l)
                r = k * LANES + l
                for j in range(DC0 // LANES):
                    js = pl.ds(j * LANES, LANES)
                    rows[r, js] = rows[r, js] * cc
            return c2
        lax.fori_loop(0, W // LANES, _scale, 0)
        pltpu.sync_copy(rows, acc.at[dstv], add=True)
        return c
    lax.fori_loop(0, NWIN, _p, 0)


def _writeback(sid, acc, den, out_h, den_h):
    for t in range(RPS // W):
        r0 = sid * RPS + t * W
        pltpu.sync_copy(acc.at[pl.ds(r0, W)], out_h.at[pl.ds(r0, W)])
    d0 = sid * RPS
    pltpu.sync_copy(den.at[pl.ds(d0, RPS)], den_h.at[pl.ds(d0, RPS)])


def _sc_pass_body(sd_h2, sA_h, dA_h, hA_h, sB_h, dB_h, hB_h,
                  oA_h, oB_h, denA_h, denB_h,
                  sdv, srcv, dstv, g1, g2, ev, rows, zden, sem,
                  acc, den):
    cid = lax.axis_index("c")
    sid = lax.axis_index("s")

    _zero_spmem(sid, rows, zden, (acc,), (den,))
    plsc.subcore_barrier()

    def _run(ss_h, sd_h, h_h, out_h, den_h):
        _msg_pass(sid, sd_h2, ss_h, sd_h, h_h,
                  sdv, srcv, dstv, g1, g2, ev, rows, sem, acc, den)
        plsc.subcore_barrier()
        _writeback(sid, acc, den, out_h, den_h)

    @pl.when(cid == 0)
    def _():
        _run(sA_h, dA_h, hA_h, oA_h, denA_h)

    @pl.when(cid == 1)
    def _():
        _run(sB_h, dB_h, hB_h, oB_h, denB_h)


def _sc_pass(srcdst, sA, dA, hA, sB, dB, hB):
    out_type = ([jax.ShapeDtypeStruct((NPAD, DC0), jnp.float32)] * 2
                + [jax.ShapeDtypeStruct((NPAD,), jnp.float32)] * 2)
    scratch = [
        pltpu.VMEM((2 * W,), jnp.int32),
        pltpu.VMEM((W,), jnp.int32),
        pltpu.VMEM((W,), jnp.int32),
        pltpu.VMEM((W,), jnp.float32),
        pltpu.VMEM((W,), jnp.float32),
        pltpu.VMEM((W,), jnp.float32),
        pltpu.VMEM((W, DC0), jnp.float32),
        pltpu.VMEM((RPS,), jnp.float32),
        pltpu.SemaphoreType.DMA,
        pltpu.VMEM_SHARED((NPAD, DC0), jnp.float32),
        pltpu.VMEM_SHARED((NPAD,), jnp.float32),
    ]
    return pl.kernel(
        _sc_pass_body,
        out_type,
        mesh=_MESH,
        scratch_types=scratch,
        name="sc_gat_pass",
    )(srcdst, sA, dA, hA, sB, dB, hB)


def _tc_layer0_body(x_ref, w0t_ref, a0t_ref, b0t_ref, attb_ref, attl_ref,
                    hb0_ref, hb1_ref, hl0_ref, hl1_ref, scal_ref):
    x = x_ref[...]
    hb = jnp.dot(x, w0t_ref[...], preferred_element_type=jnp.float32)
    hl = jnp.dot(jnp.dot(x, a0t_ref[...], preferred_element_type=jnp.float32),
                 b0t_ref[...], preferred_element_type=jnp.float32)
    hb0_ref[...] = hb[:, :128]
    hb1_ref[...] = hb[:, 128:]
    hl0_ref[...] = hl[:, :128]
    hl1_ref[...] = hl[:, 128:]
    sb = lax.dot_general(attb_ref[...], hb, (((0,), (1,)), ((), ())),
                         preferred_element_type=jnp.float32)
    sl = lax.dot_general(attl_ref[...], hl, (((0,), (1,)), ((), ())),
                         preferred_element_type=jnp.float32)
    scal_ref[...] = jnp.concatenate([sb, sl], axis=0)


def _tc_layer0(xp, w0t, a0t, b0t, attb, attl):
    return pl.pallas_call(
        _tc_layer0_body,
        grid=(GRID,),
        in_specs=[
            pl.BlockSpec((RB, 128), lambda i: (i, 0)),
            pl.BlockSpec((128, 256), lambda i: (0, 0)),
            pl.BlockSpec((128, 32), lambda i: (0, 0)),
            pl.BlockSpec((32, 256), lambda i: (0, 0)),
            pl.BlockSpec((256, 2), lambda i: (0, 0)),
            pl.BlockSpec((256, 2), lambda i: (0, 0)),
        ],
        out_specs=[
            pl.BlockSpec((RB, 128), lambda i: (i, 0)),
            pl.BlockSpec((RB, 128), lambda i: (i, 0)),
            pl.BlockSpec((RB, 128), lambda i: (i, 0)),
            pl.BlockSpec((RB, 128), lambda i: (i, 0)),
            pl.BlockSpec((4, RB), lambda i: (0, i)),
        ],
        out_shape=[
            jax.ShapeDtypeStruct((NPAD, 128), jnp.float32),
            jax.ShapeDtypeStruct((NPAD, 128), jnp.float32),
            jax.ShapeDtypeStruct((NPAD, 128), jnp.float32),
            jax.ShapeDtypeStruct((NPAD, 128), jnp.float32),
            jax.ShapeDtypeStruct((4, NPAD), jnp.float32),
        ],
    )(xp, w0t, a0t, b0t, attb, attl)


def _tc_layer1_body(mb0_ref, mb1_ref, ml0_ref, ml1_ref, denb_ref, denl_ref,
                    bias_ref, w1t_ref, a1t_ref, b1t_ref, attb_ref, attl_ref,
                    hb_ref, hl_ref, scal_ref):
    invb = (1.0 / (denb_ref[...] + 1e-16))[:, None]
    invl = (1.0 / (denl_ref[...] + 1e-16))[:, None]
    x1 = (jnp.concatenate([mb0_ref[...], mb1_ref[...]], axis=1) * invb
          + jnp.concatenate([ml0_ref[...], ml1_ref[...]], axis=1) * invl
          + bias_ref[...])
    hb = jnp.dot(x1, w1t_ref[...], preferred_element_type=jnp.float32)
    hl = jnp.dot(jnp.dot(x1, a1t_ref[...], preferred_element_type=jnp.float32),
                 b1t_ref[...], preferred_element_type=jnp.float32)
    hb_ref[...] = hb
    hl_ref[...] = hl
    sb = lax.dot_general(attb_ref[...], hb, (((0,), (1,)), ((), ())),
                         preferred_element_type=jnp.float32)
    sl = lax.dot_general(attl_ref[...], hl, (((0,), (1,)), ((), ())),
                         preferred_element_type=jnp.float32)
    scal_ref[...] = jnp.concatenate([sb, sl], axis=0)


def _tc_layer1(mb0, mb1, ml0, ml1, denb, denl, bias, w1t, a1t, b1t,
               attb, attl):
    return pl.pallas_call(
        _tc_layer1_body,
        grid=(GRID,),
        in_specs=[
            pl.BlockSpec((RB, 128), lambda i: (i, 0)),
            pl.BlockSpec((RB, 128), lambda i: (i, 0)),
            pl.BlockSpec((RB, 128), lambda i: (i, 0)),
            pl.BlockSpec((RB, 128), lambda i: (i, 0)),
            pl.BlockSpec((RB,), lambda i: (i,)),
            pl.BlockSpec((RB,), lambda i: (i,)),
            pl.BlockSpec((1, 256), lambda i: (0, 0)),
            pl.BlockSpec((256, 128), lambda i: (0, 0)),
            pl.BlockSpec((256, 32), lambda i: (0, 0)),
            pl.BlockSpec((32, 128), lambda i: (0, 0)),
            pl.BlockSpec((128, 2), lambda i: (0, 0)),
            pl.BlockSpec((128, 2), lambda i: (0, 0)),
        ],
        out_specs=[
            pl.BlockSpec((RB, 128), lambda i: (i, 0)),
            pl.BlockSpec((RB, 128), lambda i: (i, 0)),
            pl.BlockSpec((4, RB), lambda i: (0, i)),
        ],
        out_shape=[
            jax.ShapeDtypeStruct((NPAD, 128), jnp.float32),
            jax.ShapeDtypeStruct((NPAD, 128), jnp.float32),
            jax.ShapeDtypeStruct((4, NPAD), jnp.float32),
        ],
    )(mb0, mb1, ml0, ml1, denb, denl, bias, w1t, a1t, b1t, attb, attl)


def _tc_final_body(o1_ref, o2_ref, d1_ref, d2_ref, b1_ref, bl1_ref,
                   osum_ref, e1_ref, e2_ref):
    e1 = o1_ref[...] * (1.0 / (d1_ref[...] + 1e-16)) + b1_ref[...]
    e2 = o2_ref[...] * (1.0 / (d2_ref[...] + 1e-16)) + bl1_ref[...]
    e1_ref[...] = e1
    e2_ref[...] = e2
    osum_ref[...] = e1 + e2


def _tc_final(o1, o2, d1, d2, b1, bl1):
    fb = 1000
    return pl.pallas_call(
        _tc_final_body,
        grid=(N // fb,),
        in_specs=[
            pl.BlockSpec((fb, 128), lambda i: (i, 0)),
            pl.BlockSpec((fb, 128), lambda i: (i, 0)),
            pl.BlockSpec((fb, 1), lambda i: (i, 0)),
            pl.BlockSpec((fb, 1), lambda i: (i, 0)),
            pl.BlockSpec((1, 128), lambda i: (0, 0)),
            pl.BlockSpec((1, 128), lambda i: (0, 0)),
        ],
        out_specs=[
            pl.BlockSpec((fb, 128), lambda i: (i, 0)),
            pl.BlockSpec((fb, 128), lambda i: (i, 0)),
            pl.BlockSpec((fb, 128), lambda i: (i, 0)),
        ],
        out_shape=[
            jax.ShapeDtypeStruct((N, 128), jnp.float32),
            jax.ShapeDtypeStruct((N, 128), jnp.float32),
            jax.ShapeDtypeStruct((N, 128), jnp.float32),
        ],
    )(o1, o2, d1, d2, b1, bl1)


def kernel(x, edge_index, W0, att_src0, att_dst0, b0, A0, B0, al_src0,
           al_dst0, bl0, W1, att_src1, att_dst1, b1, A1, B1, al_src1,
           al_dst1, bl1):
    loop = jnp.arange(N, dtype=jnp.int32)
    npad_e = EP - E - N
    src = jnp.concatenate([edge_index[0], loop,
                           jnp.zeros((npad_e,), jnp.int32)])
    dst = jnp.concatenate([edge_index[1], loop,
                           jnp.full((npad_e,), DUMP, jnp.int32)])
    srcdst = jnp.stack([src.reshape(EP // W, W),
                        dst.reshape(EP // W, W)], axis=1).reshape(2 * EP)
    xp = jnp.pad(x, ((0, NPAD - N), (0, 0)))

    attb0 = jnp.stack([att_src0, att_dst0], axis=1)
    attl0 = jnp.stack([al_src0, al_dst0], axis=1)
    attb1 = jnp.stack([att_src1, att_dst1], axis=1)
    attl1 = jnp.stack([al_src1, al_dst1], axis=1)

    hb0, hb1, hl0, hl1, scal0 = _tc_layer0(xp, W0.T, A0.T, B0.T, attb0, attl0)
    mb0, mb1, denb, _ = _sc_pass(srcdst, scal0[0], scal0[1], hb0,
                                 scal0[0], scal0[1], hb1)
    ml0, ml1, denl, _ = _sc_pass(srcdst, scal0[2], scal0[3], hl0,
                                 scal0[2], scal0[3], hl1)

    bias0 = (b0 + bl0)[None, :]
    h1b, h1l, scal1 = _tc_layer1(mb0, mb1, ml0, ml1, denb, denl, bias0,
                                 W1.T, A1.T, B1.T, attb1, attl1)
    o1, o2, d1, d2 = _sc_pass(srcdst, scal1[0], scal1[1], h1b,
                              scal1[2], scal1[3], h1l)

    osum, emb1, emb2 = _tc_final(o1, o2, d1[:, None], d2[:, None],
                                 b1[None, :], bl1[None, :])
    return (osum, emb1, emb2)
